# baseline (speedup 1.0000x reference)
import functools

import jax
import jax.numpy as jnp
from jax import lax
from jax.experimental import pallas as pl
from jax.experimental.pallas import tpu as pltpu
from jax.experimental.pallas import tpu_sc as plsc

T = 2048
C = 768
E = 8
DFF = 3072
BLK = 256
NB = T // BLK + (E - 1)
NPAD = NB * BLK
NCHUNK = T // BLK

SC_NC = 2
SC_NS = 16
SC_NW = SC_NC * SC_NS
TPW = T // SC_NW


def _router_body(x_ref, rw_ref, rb_ref, nw_ref, nbias_ref, eps_ref,
                 pos_ref, meta_ref, oh_s, cum_s):
    xv = x_ref[...]
    logits = jnp.dot(xv, rw_ref[...], preferred_element_type=jnp.float32)
    logits = logits + rb_ref[...]
    zn = jnp.dot(xv, nw_ref[...], preferred_element_type=jnp.float32)
    zn = zn + nbias_ref[...]
    sp = jnp.maximum(zn, 0.0) + jnp.log1p(jnp.exp(-jnp.abs(zn)))
    noisy = logits + eps_ref[...] * sp

    mx = jnp.max(noisy, axis=1, keepdims=True)
    iota_e = lax.broadcasted_iota(jnp.int32, (T, E), 1).astype(jnp.float32)
    amax = jnp.min(jnp.where(noisy >= mx, iota_e, float(E)), axis=1,
                   keepdims=True)
    oh = (iota_e == amax).astype(jnp.float32)
    oh_s[...] = oh

    li = lax.broadcasted_iota(jnp.int32, (BLK, BLK), 0)
    lj = lax.broadcasted_iota(jnp.int32, (BLK, BLK), 1)
    ltri = (li > lj).astype(jnp.float32)

    def step(i, carry):
        ohc = oh_s[pl.ds(i * BLK, BLK), :]
        exc = carry + jnp.dot(ltri, ohc, preferred_element_type=jnp.float32)
        cum_s[pl.ds(i * BLK, BLK), :] = exc
        return carry + jnp.sum(ohc, axis=0, keepdims=True)

    counts = lax.fori_loop(0, NCHUNK, step,
                           jnp.zeros((1, E), jnp.float32))

    rank = jnp.sum(oh_s[...] * cum_s[...], axis=1, keepdims=True)

    nbl = jnp.floor((counts + float(BLK - 1)) / float(BLK))
    ui = lax.broadcasted_iota(jnp.int32, (E, E), 0)
    uj = lax.broadcasted_iota(jnp.int32, (E, E), 1)
    utri = (ui < uj).astype(jnp.float32)
    bstart = jnp.dot(nbl, utri, preferred_element_type=jnp.float32)
    total = jnp.sum(nbl, axis=1, keepdims=True)

    pstart = float(BLK) * bstart
    pos = jnp.sum(oh_s[...] * pstart, axis=1, keepdims=True) + rank
    pos_ref[...] = pos.astype(jnp.int32)

    bi = lax.broadcasted_iota(jnp.int32, (NB, E), 0).astype(jnp.float32)
    bcl = jnp.minimum(bi, total - 1.0)
    be = jnp.sum(jnp.where(bstart <= bcl, 1.0, 0.0), axis=1,
                 keepdims=True) - 1.0
    meta_ref[0:NB, :] = be.astype(jnp.int32)
    meta_ref[NB:NB + 1, :] = total.astype(jnp.int32)


def _router_dispatch(x2, route_W, route_b2, noise_W, noise_b2, eps2):
    return pl.pallas_call(
        _router_body,
        out_shape=(
            jax.ShapeDtypeStruct((T, 1), jnp.int32),
            jax.ShapeDtypeStruct((NB + 1, 1), jnp.int32),
        ),
        scratch_shapes=[
            pltpu.VMEM((T, E), jnp.float32),
            pltpu.VMEM((T, E), jnp.float32),
        ],
    )(x2, route_W, route_b2, noise_W, noise_b2, eps2)


def _mm(a, b):
    return lax.dot_general(a, b, (((1,), (0,)), ((), ())),
                           precision=lax.Precision.DEFAULT,
                           preferred_element_type=jnp.float32)


def _ffn_body(m_ref, xs_ref, pos_ref, w1_ref, b1_ref, w2_ref, b2_ref,
              o_ref, osb_s):
    s = pl.program_id(0)

    @pl.when(s == 0)
    def _():
        osb_s[...] = jnp.zeros((NPAD, C), jnp.bfloat16)

    @pl.when(s < m_ref[NB])
    def _():
        h = jnp.maximum(_mm(xs_ref[...], w1_ref[0]) + b1_ref[0], 0.0)
        o = _mm(h, w2_ref[0]) + b2_ref[0]
        osb_s[pl.ds(s * BLK, BLK), :] = o.astype(jnp.bfloat16)

    @pl.when(s >= NB)
    def _():
        c = s - NB
        pc = pos_ref[pl.ds(c * BLK, BLK), :]
        iota_p = lax.broadcasted_iota(jnp.int32, (BLK, NPAD), 1)
        selo = (iota_p == pc).astype(jnp.bfloat16)
        o_ref[...] = jnp.dot(selo, osb_s[...],
                             preferred_element_type=jnp.float32)


def _ffn(meta, xs, pos, W1, b1, W2, b2):
    def wmap(s, m):
        return (m[jnp.minimum(s, NB - 1)], 0, 0)

    grid_spec = pltpu.PrefetchScalarGridSpec(
        num_scalar_prefetch=1,
        grid=(NB + NCHUNK,),
        in_specs=[
            pl.BlockSpec((BLK, C),
                         lambda s, m: (jnp.minimum(s, m[NB] - 1), 0)),
            pl.BlockSpec((T, 1), lambda s, m: (0, 0)),
            pl.BlockSpec((1, C, DFF), wmap),
            pl.BlockSpec((1, 1, DFF), wmap),
            pl.BlockSpec((1, DFF, C), wmap),
            pl.BlockSpec((1, 1, C), wmap),
        ],
        out_specs=pl.BlockSpec((BLK, C),
                               lambda s, m: (jnp.maximum(s - NB, 0), 0)),
        scratch_shapes=[pltpu.VMEM((NPAD, C), jnp.bfloat16)],
    )
    return pl.pallas_call(
        _ffn_body,
        grid_spec=grid_spec,
        out_shape=jax.ShapeDtypeStruct((T, C), jnp.float32),
    )(meta, xs, pos, W1, b1.reshape(E, 1, DFF), W2, b2.reshape(E, 1, C))


@functools.cache
def _sc_kernels():
    mesh = plsc.VectorSubcoreMesh(core_axis_name="c", subcore_axis_name="s",
                                  num_cores=SC_NC)

    @functools.partial(
        pl.kernel,
        mesh=mesh,
        out_type=jax.ShapeDtypeStruct((NPAD, C), jnp.float32),
        scratch_types=[
            pltpu.VMEM((TPW,), jnp.int32),
            pltpu.VMEM((TPW, C), jnp.float32),
            pltpu.SemaphoreType.DMA,
        ],
    )
    def scatter_rows(pos_hbm, x_hbm, xs_hbm, idx_v, rows_v, sem):
        wid = lax.axis_index("s") * SC_NC + lax.axis_index("c")
        base = wid * TPW
        pltpu.sync_copy(pos_hbm.at[pl.ds(base, TPW)], idx_v)
        pltpu.sync_copy(x_hbm.at[pl.ds(base, TPW)], rows_v)
        pltpu.async_copy(rows_v, xs_hbm.at[idx_v], sem).wait()

    @functools.partial(
        pl.kernel,
        mesh=mesh,
        out_type=jax.ShapeDtypeStruct((T, C), jnp.float32),
        scratch_types=[
            pltpu.VMEM((TPW,), jnp.int32),
            pltpu.VMEM((TPW, C), jnp.float32),
            pltpu.SemaphoreType.DMA,
        ],
    )
    def gather_rows(pos_hbm, os_hbm, out_hbm, idx_v, rows_v, sem):
        wid = lax.axis_index("s") * SC_NC + lax.axis_index("c")
        base = wid * TPW
        pltpu.sync_copy(pos_hbm.at[pl.ds(base, TPW)], idx_v)
        pltpu.async_copy(os_hbm.at[idx_v], rows_v, sem).wait()
        pltpu.sync_copy(rows_v, out_hbm.at[pl.ds(base, TPW)])

    return scatter_rows, gather_rows


def kernel(x, route_W, route_b, noise_W, noise_b, W1, b1, W2, b2, eps):
    x2 = x.reshape(T, C)
    eps2 = eps.reshape(T, E)
    rb2 = route_b.reshape(1, E)
    nb2 = noise_b.reshape(1, E)

    pos, meta = _router_dispatch(x2, route_W, rb2, noise_W, nb2, eps2)
    pos1 = pos.reshape(T)
    meta1 = meta.reshape(NB + 1)

    scatter_rows, gather_rows = _sc_kernels()
    xs = scatter_rows(pos1, x2)
    out = _ffn(meta1, xs, pos, W1, b1, W2, b2)
    return out.reshape(1, T, C)


# speedup vs baseline: 1.1707x; 1.1707x over previous
import functools

import jax
import jax.numpy as jnp
from jax import lax
from jax.experimental import pallas as pl
from jax.experimental.pallas import tpu as pltpu
from jax.experimental.pallas import tpu_sc as plsc

T = 2048
C = 768
E = 8
DFF = 3072
BLK = 512
NB = T // BLK + (E - 1)
NPAD = NB * BLK
NCHUNK = T // BLK

SC_NC = 2
SC_NS = 16
SC_NW = SC_NC * SC_NS
TPW = T // SC_NW


def _router_body(x_ref, rw_ref, rb_ref, nw_ref, nbias_ref, eps_ref,
                 pos_ref, meta_ref, oh_s, cum_s):
    xv = x_ref[...]
    logits = jnp.dot(xv, rw_ref[...], preferred_element_type=jnp.float32)
    logits = logits + rb_ref[...]
    zn = jnp.dot(xv, nw_ref[...], preferred_element_type=jnp.float32)
    zn = zn + nbias_ref[...]
    sp = jnp.maximum(zn, 0.0) + jnp.log1p(jnp.exp(-jnp.abs(zn)))
    noisy = logits + eps_ref[...] * sp

    mx = jnp.max(noisy, axis=1, keepdims=True)
    iota_e = lax.broadcasted_iota(jnp.int32, (T, E), 1).astype(jnp.float32)
    amax = jnp.min(jnp.where(noisy >= mx, iota_e, float(E)), axis=1,
                   keepdims=True)
    oh = (iota_e == amax).astype(jnp.float32)
    oh_s[...] = oh

    li = lax.broadcasted_iota(jnp.int32, (BLK, BLK), 0)
    lj = lax.broadcasted_iota(jnp.int32, (BLK, BLK), 1)
    ltri = (li > lj).astype(jnp.float32)

    def step(i, carry):
        ohc = oh_s[pl.ds(i * BLK, BLK), :]
        exc = carry + jnp.dot(ltri, ohc, preferred_element_type=jnp.float32)
        cum_s[pl.ds(i * BLK, BLK), :] = exc
        return carry + jnp.sum(ohc, axis=0, keepdims=True)

    counts = lax.fori_loop(0, NCHUNK, step,
                           jnp.zeros((1, E), jnp.float32))

    rank = jnp.sum(oh_s[...] * cum_s[...], axis=1, keepdims=True)

    nbl = jnp.floor((counts + float(BLK - 1)) / float(BLK))
    ui = lax.broadcasted_iota(jnp.int32, (E, E), 0)
    uj = lax.broadcasted_iota(jnp.int32, (E, E), 1)
    utri = (ui < uj).astype(jnp.float32)
    bstart = jnp.dot(nbl, utri, preferred_element_type=jnp.float32)
    total = jnp.sum(nbl, axis=1, keepdims=True)

    pstart = float(BLK) * bstart
    pos = jnp.sum(oh_s[...] * pstart, axis=1, keepdims=True) + rank
    pos_ref[...] = pos.astype(jnp.int32)

    bi = lax.broadcasted_iota(jnp.int32, (NB, E), 0).astype(jnp.float32)
    bcl = jnp.minimum(bi, total - 1.0)
    be = jnp.sum(jnp.where(bstart <= bcl, 1.0, 0.0), axis=1,
                 keepdims=True) - 1.0
    meta_ref[0:NB, :] = be.astype(jnp.int32)
    meta_ref[NB:NB + 1, :] = total.astype(jnp.int32)


def _router_dispatch(x2, route_W, route_b2, noise_W, noise_b2, eps2):
    return pl.pallas_call(
        _router_body,
        out_shape=(
            jax.ShapeDtypeStruct((T, 1), jnp.int32),
            jax.ShapeDtypeStruct((NB + 1, 1), jnp.int32),
        ),
        scratch_shapes=[
            pltpu.VMEM((T, E), jnp.float32),
            pltpu.VMEM((T, E), jnp.float32),
        ],
    )(x2, route_W, route_b2, noise_W, noise_b2, eps2)


def _mm(a, b):
    return lax.dot_general(a, b, (((1,), (0,)), ((), ())),
                           precision=lax.Precision.DEFAULT,
                           preferred_element_type=jnp.float32)


def _ffn_body(m_ref, xs_ref, w1_ref, b1_ref, w2_ref, b2_ref, o_ref):
    b = pl.program_id(0)

    @pl.when(b < m_ref[NB])
    def _():
        h = jnp.maximum(_mm(xs_ref[...], w1_ref[0]) + b1_ref[0], 0.0)
        o_ref[...] = _mm(h, w2_ref[0]) + b2_ref[0]


def _ffn(meta, xs, W1, b1, W2, b2):
    def wmap(b, m):
        return (m[b], 0, 0)

    grid_spec = pltpu.PrefetchScalarGridSpec(
        num_scalar_prefetch=1,
        grid=(NB,),
        in_specs=[
            pl.BlockSpec((BLK, C),
                         lambda b, m: (jnp.minimum(b, m[NB] - 1), 0)),
            pl.BlockSpec((1, C, DFF), wmap),
            pl.BlockSpec((1, 1, DFF), wmap),
            pl.BlockSpec((1, DFF, C), wmap),
            pl.BlockSpec((1, 1, C), wmap),
        ],
        out_specs=pl.BlockSpec((BLK, C), lambda b, m: (b, 0)),
    )
    return pl.pallas_call(
        _ffn_body,
        grid_spec=grid_spec,
        out_shape=jax.ShapeDtypeStruct((NPAD, C), jnp.float32),
    )(meta, xs, W1, b1.reshape(E, 1, DFF), W2, b2.reshape(E, 1, C))


@functools.cache
def _sc_kernels():
    mesh = plsc.VectorSubcoreMesh(core_axis_name="c", subcore_axis_name="s",
                                  num_cores=SC_NC)

    @functools.partial(
        pl.kernel,
        mesh=mesh,
        out_type=jax.ShapeDtypeStruct((NPAD, C), jnp.float32),
        scratch_types=[
            pltpu.VMEM((TPW,), jnp.int32),
            pltpu.VMEM((TPW, C), jnp.float32),
            pltpu.SemaphoreType.DMA,
        ],
    )
    def scatter_rows(pos_hbm, x_hbm, xs_hbm, idx_v, rows_v, sem):
        wid = lax.axis_index("s") * SC_NC + lax.axis_index("c")
        base = wid * TPW
        pltpu.sync_copy(pos_hbm.at[pl.ds(base, TPW)], idx_v)
        pltpu.sync_copy(x_hbm.at[pl.ds(base, TPW)], rows_v)
        pltpu.async_copy(rows_v, xs_hbm.at[idx_v], sem).wait()

    @functools.partial(
        pl.kernel,
        mesh=mesh,
        out_type=jax.ShapeDtypeStruct((T, C), jnp.float32),
        scratch_types=[
            pltpu.VMEM((TPW,), jnp.int32),
            pltpu.VMEM((TPW, C), jnp.float32),
            pltpu.SemaphoreType.DMA,
        ],
    )
    def gather_rows(pos_hbm, os_hbm, out_hbm, idx_v, rows_v, sem):
        wid = lax.axis_index("s") * SC_NC + lax.axis_index("c")
        base = wid * TPW
        pltpu.sync_copy(pos_hbm.at[pl.ds(base, TPW)], idx_v)
        pltpu.async_copy(os_hbm.at[idx_v], rows_v, sem).wait()
        pltpu.sync_copy(rows_v, out_hbm.at[pl.ds(base, TPW)])

    return scatter_rows, gather_rows


def kernel(x, route_W, route_b, noise_W, noise_b, W1, b1, W2, b2, eps):
    x2 = x.reshape(T, C)
    eps2 = eps.reshape(T, E)
    rb2 = route_b.reshape(1, E)
    nb2 = noise_b.reshape(1, E)

    pos, meta = _router_dispatch(x2, route_W, rb2, noise_W, nb2, eps2)
    pos1 = pos.reshape(T)
    meta1 = meta.reshape(NB + 1)

    scatter_rows, gather_rows = _sc_kernels()
    xs = scatter_rows(pos1, x2)
    os = _ffn(meta1, xs, W1, b1, W2, b2)
    out = gather_rows(pos1, os)
    return out.reshape(1, T, C)


# speedup vs baseline: 1.1709x; 1.0002x over previous
import functools

import jax
import jax.numpy as jnp
from jax import lax
from jax.experimental import pallas as pl
from jax.experimental.pallas import tpu as pltpu
from jax.experimental.pallas import tpu_sc as plsc

T = 2048
C = 768
E = 8
DFF = 3072
BLK = 512
NB = T // BLK + (E - 1)
NPAD = NB * BLK
NCHUNK = T // BLK

SC_NC = 2
SC_NS = 16
SC_NW = SC_NC * SC_NS
TPW = T // SC_NW


def _router_body(x_ref, rw_ref, rb_ref, nw_ref, nbias_ref, eps_ref,
                 pos_ref, meta_ref, oh_s, cum_s):
    xv = x_ref[...]
    logits = jnp.dot(xv, rw_ref[...], preferred_element_type=jnp.float32)
    logits = logits + rb_ref[...]
    zn = jnp.dot(xv, nw_ref[...], preferred_element_type=jnp.float32)
    zn = zn + nbias_ref[...]
    sp = jnp.maximum(zn, 0.0) + jnp.log1p(jnp.exp(-jnp.abs(zn)))
    noisy = logits + eps_ref[...] * sp

    mx = jnp.max(noisy, axis=1, keepdims=True)
    iota_e = lax.broadcasted_iota(jnp.int32, (T, E), 1).astype(jnp.float32)
    amax = jnp.min(jnp.where(noisy >= mx, iota_e, float(E)), axis=1,
                   keepdims=True)
    oh = (iota_e == amax).astype(jnp.float32)
    oh_s[...] = oh

    li = lax.broadcasted_iota(jnp.int32, (BLK, BLK), 0)
    lj = lax.broadcasted_iota(jnp.int32, (BLK, BLK), 1)
    ltri = (li > lj).astype(jnp.float32)

    def step(i, carry):
        ohc = oh_s[pl.ds(i * BLK, BLK), :]
        exc = carry + jnp.dot(ltri, ohc, preferred_element_type=jnp.float32)
        cum_s[pl.ds(i * BLK, BLK), :] = exc
        return carry + jnp.sum(ohc, axis=0, keepdims=True)

    counts = lax.fori_loop(0, NCHUNK, step,
                           jnp.zeros((1, E), jnp.float32))

    rank = jnp.sum(oh_s[...] * cum_s[...], axis=1, keepdims=True)

    nbl = jnp.floor((counts + float(BLK - 1)) / float(BLK))
    ui = lax.broadcasted_iota(jnp.int32, (E, E), 0)
    uj = lax.broadcasted_iota(jnp.int32, (E, E), 1)
    utri = (ui < uj).astype(jnp.float32)
    bstart = jnp.dot(nbl, utri, preferred_element_type=jnp.float32)
    total = jnp.sum(nbl, axis=1, keepdims=True)

    pstart = float(BLK) * bstart
    pos = jnp.sum(oh_s[...] * pstart, axis=1, keepdims=True) + rank
    pos_ref[...] = pos.astype(jnp.int32)

    bi = lax.broadcasted_iota(jnp.int32, (NB, E), 0).astype(jnp.float32)
    bcl = jnp.minimum(bi, total - 1.0)
    be = jnp.sum(jnp.where(bstart <= bcl, 1.0, 0.0), axis=1,
                 keepdims=True) - 1.0
    meta_ref[0:NB, :] = be.astype(jnp.int32)
    meta_ref[NB:NB + 1, :] = total.astype(jnp.int32)


def _router_dispatch(x2, route_W, route_b2, noise_W, noise_b2, eps2):
    return pl.pallas_call(
        _router_body,
        out_shape=(
            jax.ShapeDtypeStruct((T, 1), jnp.int32),
            jax.ShapeDtypeStruct((NB + 1, 1), jnp.int32),
        ),
        scratch_shapes=[
            pltpu.VMEM((T, E), jnp.float32),
            pltpu.VMEM((T, E), jnp.float32),
        ],
    )(x2, route_W, route_b2, noise_W, noise_b2, eps2)


def _mm(a, b):
    return lax.dot_general(a, b, (((1,), (0,)), ((), ())),
                           precision=lax.Precision.DEFAULT,
                           preferred_element_type=jnp.float32)


def _ffn_body(m_ref, xs_ref, w1_ref, b1_ref, w2_ref, b2_ref, o_ref):
    b = pl.program_id(0)

    @pl.when(b < m_ref[NB])
    def _():
        e = m_ref[b]
        h = jnp.maximum(_mm(xs_ref[...], w1_ref[0]) + b1_ref[pl.ds(e, 1), :],
                        0.0)
        o_ref[...] = _mm(h, w2_ref[0]) + b2_ref[pl.ds(e, 1), :]


def _ffn(meta, xs, W1, b1, W2, b2):
    def wmap(b, m):
        return (m[b], 0, 0)

    grid_spec = pltpu.PrefetchScalarGridSpec(
        num_scalar_prefetch=1,
        grid=(NB,),
        in_specs=[
            pl.BlockSpec((BLK, C),
                         lambda b, m: (jnp.minimum(b, m[NB] - 1), 0)),
            pl.BlockSpec((1, C, DFF), wmap),
            pl.BlockSpec((E, DFF), lambda b, m: (0, 0)),
            pl.BlockSpec((1, DFF, C), wmap),
            pl.BlockSpec((E, C), lambda b, m: (0, 0)),
        ],
        out_specs=pl.BlockSpec((BLK, C), lambda b, m: (b, 0)),
    )
    return pl.pallas_call(
        _ffn_body,
        grid_spec=grid_spec,
        out_shape=jax.ShapeDtypeStruct((NPAD, C), jnp.float32),
    )(meta, xs, W1, b1, W2, b2)


@functools.cache
def _sc_kernels():
    mesh = plsc.VectorSubcoreMesh(core_axis_name="c", subcore_axis_name="s",
                                  num_cores=SC_NC)

    @functools.partial(
        pl.kernel,
        mesh=mesh,
        out_type=jax.ShapeDtypeStruct((NPAD, C), jnp.float32),
        scratch_types=[
            pltpu.VMEM((TPW,), jnp.int32),
            pltpu.VMEM((TPW, C), jnp.float32),
            pltpu.SemaphoreType.DMA,
        ],
    )
    def scatter_rows(pos_hbm, x_hbm, xs_hbm, idx_v, rows_v, sem):
        wid = lax.axis_index("s") * SC_NC + lax.axis_index("c")
        base = wid * TPW
        pltpu.sync_copy(pos_hbm.at[pl.ds(base, TPW)], idx_v)
        pltpu.sync_copy(x_hbm.at[pl.ds(base, TPW)], rows_v)
        pltpu.async_copy(rows_v, xs_hbm.at[idx_v], sem).wait()

    @functools.partial(
        pl.kernel,
        mesh=mesh,
        out_type=jax.ShapeDtypeStruct((T, C), jnp.float32),
        scratch_types=[
            pltpu.VMEM((TPW,), jnp.int32),
            pltpu.VMEM((TPW, C), jnp.float32),
            pltpu.SemaphoreType.DMA,
        ],
    )
    def gather_rows(pos_hbm, os_hbm, out_hbm, idx_v, rows_v, sem):
        wid = lax.axis_index("s") * SC_NC + lax.axis_index("c")
        base = wid * TPW
        pltpu.sync_copy(pos_hbm.at[pl.ds(base, TPW)], idx_v)
        pltpu.async_copy(os_hbm.at[idx_v], rows_v, sem).wait()
        pltpu.sync_copy(rows_v, out_hbm.at[pl.ds(base, TPW)])

    return scatter_rows, gather_rows


def kernel(x, route_W, route_b, noise_W, noise_b, W1, b1, W2, b2, eps):
    x2 = x.reshape(T, C)
    eps2 = eps.reshape(T, E)
    rb2 = route_b.reshape(1, E)
    nb2 = noise_b.reshape(1, E)

    pos, meta = _router_dispatch(x2, route_W, rb2, noise_W, nb2, eps2)
    pos1 = pos.reshape(T)
    meta1 = meta.reshape(NB + 1)

    scatter_rows, gather_rows = _sc_kernels()
    xs = scatter_rows(pos1, x2)
    os = _ffn(meta1, xs, W1, b1, W2, b2)
    out = gather_rows(pos1, os)
    return out.reshape(1, T, C)
